# baseline (speedup 1.0000x reference)
import functools

import jax
import jax.numpy as jnp
from jax import lax
from jax.experimental import pallas as pl
from jax.experimental.pallas import tpu as pltpu
from jax.experimental.pallas import tpu_sc as plsc

_B = 16384
_D = 64
_NT = 5
_CHUNK = 128
_NBUF = 4


@functools.cache
def _build():
    info = plsc.get_sparse_core_info()
    nc, ns = info.num_cores, info.num_subcores
    nw = nc * ns
    b_per_w = _B // nw
    n_chunks = b_per_w // _CHUNK
    n_tasks = _NT * n_chunks
    mesh = plsc.VectorSubcoreMesh(core_axis_name="c", subcore_axis_name="s")

    @functools.partial(
        pl.kernel,
        mesh=mesh,
        out_type=jax.ShapeDtypeStruct((_B, _NT * _D), jnp.float32),
        compiler_params=pltpu.CompilerParams(use_tc_tiling_on_sc=False),
        scratch_types=(
            [pltpu.VMEM((n_tasks, _CHUNK), jnp.int32)]
            + [pltpu.VMEM((_CHUNK, _D), jnp.float32) for _ in range(_NBUF)]
            + [pltpu.SemaphoreType.DMA for _ in range(_NBUF)]
        ),
    )
    def node_embedding(idx_h, w_cat, w_sub, w_elem, w_brand, w_item, out_h,
                       idx_v, *bufs_and_sems):
        rows = bufs_and_sems[:_NBUF]
        sems = bufs_and_sems[_NBUF:]
        tabs = [w_cat, w_sub, w_elem, w_brand, w_item]
        wid = lax.axis_index("s") * nc + lax.axis_index("c")
        base = wid * b_per_w

        pltpu.sync_copy(idx_h.at[wid], idx_v)

        def start(i):
            t = i // n_chunks
            return pltpu.async_copy(
                tabs[t].at[idx_v.at[i]], rows[i % _NBUF], sems[i % _NBUF])

        inflight = [None] * _NBUF
        for i in range(min(_NBUF, n_tasks)):
            inflight[i % _NBUF] = start(i)
        for i in range(n_tasks):
            t, c = i // n_chunks, i % n_chunks
            inflight[i % _NBUF].wait()
            pltpu.sync_copy(
                rows[i % _NBUF],
                out_h.at[pl.ds(base + c * _CHUNK, _CHUNK), pl.ds(t * _D, _D)])
            j = i + _NBUF
            if j < n_tasks:
                inflight[j % _NBUF] = start(j)

    return node_embedding, nw, n_chunks


def kernel(categories, sub_categories, elements, brands, product_id_remapped,
           W_cat, W_sub, W_elem, W_brand, W_item):
    fn, nw, n_chunks = _build()
    idx = jnp.stack([categories, sub_categories, elements, brands,
                     product_id_remapped]).astype(jnp.int32)
    idx = idx.reshape(_NT, nw, n_chunks, _CHUNK).transpose(1, 0, 2, 3)
    idx = idx.reshape(nw, _NT * n_chunks, _CHUNK)
    return fn(idx, W_cat, W_sub, W_elem, W_brand, W_item)
